# baseline (speedup 1.0000x reference)
import functools
import math

import jax
import jax.numpy as jnp
from jax import lax
from jax.experimental import pallas as pl
from jax.experimental.pallas import tpu as pltpu
from jax.experimental.pallas import tpu_sc as plsc

D_MODEL = 768
SCALE = math.sqrt(float(D_MODEL))
LANES = 16


@functools.cache
def _make_kernel(SEQ, BATCH, V, D):
    B = SEQ * BATCH
    info = plsc.get_sparse_core_info()
    NC, NS = info.num_cores, info.num_subcores
    NW = NC * NS
    b_per_w = B // NW
    CH = 32
    n_ch = b_per_w // CH
    CH_S = CH // BATCH
    NT = D // 128
    NJ = 128 // LANES

    mesh = plsc.VectorSubcoreMesh(core_axis_name="c", subcore_axis_name="s")

    @functools.partial(
        pl.kernel,
        mesh=mesh,
        out_type=jax.ShapeDtypeStruct((SEQ, NT, BATCH, 128), jnp.float32),
        scratch_types=[
            pltpu.VMEM((2, CH), jnp.int32),
            pltpu.VMEM((CH, D), jnp.float32),
            pltpu.VMEM((CH, D), jnp.float32),
            pltpu.VMEM((CH_S, NT, BATCH, 128), jnp.float32),
            pltpu.VMEM((CH_S, NT, BATCH, 128), jnp.float32),
            pltpu.VMEM((CH_S * D,), jnp.float32),
            pltpu.VMEM((CH_S * D,), jnp.float32),
            pltpu.SemaphoreType.DMA,
            pltpu.SemaphoreType.DMA,
            pltpu.SemaphoreType.DMA,
            pltpu.SemaphoreType.DMA,
            pltpu.SemaphoreType.DMA,
            pltpu.SemaphoreType.DMA,
        ],
    )
    def k(x_hbm, table_hbm, pe_hbm, out_hbm,
          idx_v, rows0, rows1, ov0, ov1, pe0, pe1,
          sg0, sg1, sp0, sp1, so0, so1):
        rows = [rows0, rows1]
        outv = [ov0, ov1]
        pev = [pe0, pe1]
        sg = [sg0, sg1]
        sp = [sp0, sp1]
        so = [so0, so1]
        wid = lax.axis_index("s") * NC + lax.axis_index("c")
        base = wid * b_per_w

        gathers = [None, None]
        pes = [None, None]
        outs = [None, None]

        def start(c):
            b = c & 1
            cbase = pl.multiple_of(base + c * CH, CH)
            pltpu.sync_copy(x_hbm.at[pl.ds(cbase, CH)], idx_v.at[b])
            gathers[b] = pltpu.async_copy(table_hbm.at[idx_v.at[b]], rows[b], sg[b])
            pes[b] = pltpu.async_copy(
                pe_hbm.at[pl.ds(pl.multiple_of((cbase // BATCH) * D, CH_S * D),
                                CH_S * D)],
                pev[b], sp[b],
            )

        def compute(rows_v, pe_v, out_v):
            def s_body(si, carry2):
                def d_body(di, carry3):
                    doff = di * LANES
                    ti = di >> 3
                    jcol = (di & 7) * LANES
                    pvec = pe_v[pl.ds(si * D + doff, LANES)]
                    for b in range(BATCH):
                        out_v[si, ti, b, pl.ds(jcol, LANES)] = (
                            rows_v[si * BATCH + b, pl.ds(doff, LANES)] * SCALE
                            + pvec
                        )
                    return carry3

                return lax.fori_loop(0, NT * NJ, d_body, carry2)

            lax.fori_loop(0, CH_S, s_body, 0)

        start(0)
        for c in range(n_ch):
            b = c & 1
            if c + 1 < n_ch:
                start(c + 1)
            gathers[b].wait()
            pes[b].wait()
            if c >= 2:
                outs[b].wait()
            compute(rows[b], pev[b], outv[b])
            sbase = pl.multiple_of((base + c * CH) // BATCH, CH_S)
            outs[b] = pltpu.async_copy(outv[b], out_hbm.at[pl.ds(sbase, CH_S)], so[b])
        outs[(n_ch - 2) & 1].wait()
        outs[(n_ch - 1) & 1].wait()

    return k


def kernel(x, table, pe):
    seq, batch = x.shape
    B = seq * batch
    xf = x.reshape(B)
    pef = pe.reshape(pe.shape[0] * pe.shape[2])
    out4 = _make_kernel(seq, batch, table.shape[0], table.shape[1])(xf, table, pef)
    return out4.transpose(0, 2, 1, 3).reshape(seq, batch, table.shape[1])


# speedup vs baseline: 2.5416x; 2.5416x over previous
import functools
import math

import jax
import jax.numpy as jnp
from jax import lax
from jax.experimental import pallas as pl
from jax.experimental.pallas import tpu as pltpu
from jax.experimental.pallas import tpu_sc as plsc

D_MODEL = 768
SCALE = math.sqrt(float(D_MODEL))
LANES = 16


@functools.cache
def _make_kernel(SEQ, BATCH, V, D):
    B = SEQ * BATCH
    info = plsc.get_sparse_core_info()
    NC, NS = info.num_cores, info.num_subcores
    NW = NC * NS
    b_per_w = B // NW
    CH = 32
    n_ch = b_per_w // CH
    CH_S = CH // BATCH
    NT = D // 128
    NJ = 128 // LANES

    mesh = plsc.VectorSubcoreMesh(core_axis_name="c", subcore_axis_name="s")

    @functools.partial(
        pl.kernel,
        mesh=mesh,
        out_type=jax.ShapeDtypeStruct((SEQ, NT, BATCH, 128), jnp.float32),
        scratch_types=[
            pltpu.VMEM((2, CH), jnp.int32),
            pltpu.VMEM((CH, D), jnp.float32),
            pltpu.VMEM((CH, D), jnp.float32),
            pltpu.VMEM((CH_S, NT, BATCH, 128), jnp.float32),
            pltpu.VMEM((CH_S, NT, BATCH, 128), jnp.float32),
            pltpu.VMEM((CH_S * D,), jnp.float32),
            pltpu.VMEM((CH_S * D,), jnp.float32),
            pltpu.SemaphoreType.DMA,
            pltpu.SemaphoreType.DMA,
            pltpu.SemaphoreType.DMA,
            pltpu.SemaphoreType.DMA,
            pltpu.SemaphoreType.DMA,
            pltpu.SemaphoreType.DMA,
        ],
    )
    def k(x_hbm, table_hbm, pe_hbm, out_hbm,
          idx_v, rows0, rows1, ov0, ov1, pe0, pe1,
          sg0, sg1, sp0, sp1, so0, so1):
        rows = [rows0, rows1]
        outv = [ov0, ov1]
        pev = [pe0, pe1]
        sg = [sg0, sg1]
        sp = [sp0, sp1]
        so = [so0, so1]
        wid = lax.axis_index("s") * NC + lax.axis_index("c")
        base = wid * b_per_w

        gathers = [None, None]
        pes = [None, None]
        outs = [None, None]

        def start(c):
            b = c & 1
            cbase = pl.multiple_of(base + c * CH, CH)
            pltpu.sync_copy(x_hbm.at[pl.ds(cbase, CH)], idx_v.at[b])
            gathers[b] = pltpu.async_copy(table_hbm.at[idx_v.at[b]], rows[b], sg[b])
            pes[b] = pltpu.async_copy(
                pe_hbm.at[pl.ds(pl.multiple_of((cbase // BATCH) * D, CH_S * D),
                                CH_S * D)],
                pev[b], sp[b],
            )

        def compute(rows_v, pe_v, out_v):
            def s_body(si, carry2):
                @plsc.parallel_loop(0, NT * NJ, unroll=4)
                def d_body(di):
                    doff = di * LANES
                    ti = di >> 3
                    jcol = (di & 7) * LANES
                    pvec = pe_v[pl.ds(si * D + doff, LANES)]
                    for b in range(BATCH):
                        out_v[si, ti, b, pl.ds(jcol, LANES)] = (
                            rows_v[si * BATCH + b, pl.ds(doff, LANES)] * SCALE
                            + pvec
                        )

                return carry2

            lax.fori_loop(0, CH_S, s_body, 0)

        start(0)
        for c in range(n_ch):
            b = c & 1
            if c + 1 < n_ch:
                start(c + 1)
            gathers[b].wait()
            pes[b].wait()
            if c >= 2:
                outs[b].wait()
            compute(rows[b], pev[b], outv[b])
            sbase = pl.multiple_of((base + c * CH) // BATCH, CH_S)
            outs[b] = pltpu.async_copy(outv[b], out_hbm.at[pl.ds(sbase, CH_S)], so[b])
        outs[(n_ch - 2) & 1].wait()
        outs[(n_ch - 1) & 1].wait()

    return k


def kernel(x, table, pe):
    seq, batch = x.shape
    B = seq * batch
    xf = x.reshape(B)
    pef = pe.reshape(pe.shape[0] * pe.shape[2])
    out4 = _make_kernel(seq, batch, table.shape[0], table.shape[1])(xf, table, pef)
    return out4.transpose(0, 2, 1, 3).reshape(seq, batch, table.shape[1])


# speedup vs baseline: 2.7614x; 1.0864x over previous
import functools
import math

import jax
import jax.numpy as jnp
from jax import lax
from jax.experimental import pallas as pl
from jax.experimental.pallas import tpu as pltpu
from jax.experimental.pallas import tpu_sc as plsc

D_MODEL = 768
SCALE = math.sqrt(float(D_MODEL))
LANES = 16
TILE_W = 128


@functools.cache
def _make_kernel(SEQ, BATCH, V, D):
    B = SEQ * BATCH
    info = plsc.get_sparse_core_info()
    NC, NS = info.num_cores, info.num_subcores
    NW = NC * NS
    b_per_w = B // NW
    CH = 32
    n_ch = b_per_w // CH
    CH_S = CH // BATCH
    NT = D // TILE_W
    NJ = TILE_W // LANES

    mesh = plsc.VectorSubcoreMesh(core_axis_name="c", subcore_axis_name="s")

    @functools.partial(
        pl.kernel,
        mesh=mesh,
        out_type=jax.ShapeDtypeStruct((SEQ, NT, BATCH, TILE_W), jnp.float32),
        scratch_types=[
            pltpu.VMEM((2, CH), jnp.int32),
            pltpu.VMEM((CH, D), jnp.float32),
            pltpu.VMEM((CH, D), jnp.float32),
            pltpu.VMEM((CH_S, NT, BATCH, TILE_W), jnp.float32),
            pltpu.VMEM((CH_S, NT, BATCH, TILE_W), jnp.float32),
            pltpu.VMEM((CH_S * D,), jnp.float32),
            pltpu.VMEM((CH_S * D,), jnp.float32),
            pltpu.SemaphoreType.DMA,
            pltpu.SemaphoreType.DMA,
            pltpu.SemaphoreType.DMA,
            pltpu.SemaphoreType.DMA,
            pltpu.SemaphoreType.DMA,
            pltpu.SemaphoreType.DMA,
        ],
    )
    def k(x_hbm, table_hbm, pe_hbm, out_hbm,
          idx_v, rows0, rows1, ov0, ov1, pe0, pe1,
          sg0, sg1, sp0, sp1, so0, so1):
        rows = [rows0, rows1]
        outv = [ov0, ov1]
        pev = [pe0, pe1]
        sg = [sg0, sg1]
        sp = [sp0, sp1]
        so = [so0, so1]
        wid = lax.axis_index("s") * NC + lax.axis_index("c")
        base = wid * b_per_w

        def start(c, p):
            cbase = pl.multiple_of(base + c * CH, CH)
            pltpu.sync_copy(x_hbm.at[pl.ds(cbase, CH)], idx_v.at[p])
            pltpu.make_async_copy(
                table_hbm.at[idx_v.at[p]], rows[p], sg[p]
            ).start()
            pltpu.make_async_copy(
                pe_hbm.at[pl.ds(pl.multiple_of((cbase // BATCH) * D, CH_S * D),
                                CH_S * D)],
                pev[p], sp[p],
            ).start()

        def wait_in(p):
            pltpu.make_async_copy(
                table_hbm.at[idx_v.at[p]], rows[p], sg[p]
            ).wait()
            pltpu.make_async_copy(
                pe_hbm.at[pl.ds(0, CH_S * D)], pev[p], sp[p]
            ).wait()

        def wait_out(p):
            pltpu.make_async_copy(
                outv[p], out_hbm.at[pl.ds(0, CH_S)], so[p]
            ).wait()

        def writeback(c, p):
            sbase = pl.multiple_of((base + c * CH) // BATCH, CH_S)
            pltpu.make_async_copy(
                outv[p], out_hbm.at[pl.ds(sbase, CH_S)], so[p]
            ).start()

        def compute(rows_v, pe_v, out_v):
            def s_body(si, carry2):
                @plsc.parallel_loop(0, NT * NJ, unroll=8)
                def d_body(di):
                    doff = di * LANES
                    ti = di >> 3
                    jcol = (di & 7) * LANES
                    pvec = pe_v[pl.ds(si * D + doff, LANES)]
                    for b in range(BATCH):
                        out_v[si, ti, b, pl.ds(jcol, LANES)] = (
                            rows_v[si * BATCH + b, pl.ds(doff, LANES)] * SCALE
                            + pvec
                        )

                return carry2

            lax.fori_loop(0, CH_S, s_body, 0)

        start(0, 0)
        for c in (0, 1):
            p = c & 1
            start(c + 1, 1 - p)
            wait_in(p)
            compute(rows[p], pev[p], outv[p])
            writeback(c, p)

        def pair_body(i, carry):
            c0 = i * 2
            for p in (0, 1):
                c = c0 + p

                @pl.when(c + 1 < n_ch)
                def _():
                    start(c + 1, 1 - p)

                wait_in(p)
                wait_out(p)
                compute(rows[p], pev[p], outv[p])
                writeback(c, p)
            return carry

        lax.fori_loop(1, n_ch // 2, pair_body, 0)
        wait_out(0)
        wait_out(1)

    return k


def kernel(x, table, pe):
    seq, batch = x.shape
    B = seq * batch
    D = table.shape[1]
    xf = x.reshape(B)
    pef = pe.reshape(pe.shape[0] * pe.shape[2])
    out4 = _make_kernel(seq, batch, table.shape[0], D)(xf, table, pef)
    return out4.transpose(0, 2, 1, 3).reshape(seq, batch, D)


# speedup vs baseline: 2.8130x; 1.0187x over previous
import functools
import math

import jax
import jax.numpy as jnp
from jax import lax
from jax.experimental import pallas as pl
from jax.experimental.pallas import tpu as pltpu
from jax.experimental.pallas import tpu_sc as plsc

D_MODEL = 768
SCALE = math.sqrt(float(D_MODEL))
LANES = 16
TILE_W = 128


@functools.cache
def _make_kernel(SEQ, BATCH, V, D):
    B = SEQ * BATCH
    info = plsc.get_sparse_core_info()
    NC, NS = info.num_cores, info.num_subcores
    NW = NC * NS
    b_per_w = B // NW
    s_per_w = b_per_w // BATCH
    CH = 32
    n_ch = b_per_w // CH
    CH_S = CH // BATCH
    NT = D // TILE_W
    NJ = TILE_W // LANES

    mesh = plsc.VectorSubcoreMesh(core_axis_name="c", subcore_axis_name="s")

    @functools.partial(
        pl.kernel,
        mesh=mesh,
        out_type=jax.ShapeDtypeStruct((SEQ, NT, BATCH, TILE_W), jnp.float32),
        scratch_types=[
            pltpu.VMEM((BATCH, s_per_w), jnp.int32),
            pltpu.VMEM((CH, D), jnp.float32),
            pltpu.VMEM((CH, D), jnp.float32),
            pltpu.VMEM((CH_S, NT, BATCH, TILE_W), jnp.float32),
            pltpu.VMEM((CH_S, NT, BATCH, TILE_W), jnp.float32),
            pltpu.VMEM((CH_S * D,), jnp.float32),
            pltpu.VMEM((CH_S * D,), jnp.float32),
            pltpu.SemaphoreType.DMA,
            pltpu.SemaphoreType.DMA,
            pltpu.SemaphoreType.DMA,
            pltpu.SemaphoreType.DMA,
            pltpu.SemaphoreType.DMA,
            pltpu.SemaphoreType.DMA,
        ],
    )
    def k(xt_hbm, table_hbm, pe_hbm, out_hbm,
          idx_bm, rows0, rows1, ov0, ov1, pe0, pe1,
          sg0, sg1, sp0, sp1, so0, so1):
        rows = [rows0, rows1]
        outv = [ov0, ov1]
        pev = [pe0, pe1]
        sg = [sg0, sg1]
        sp = [sp0, sp1]
        so = [so0, so1]
        wid = lax.axis_index("s") * NC + lax.axis_index("c")
        base = wid * b_per_w

        pltpu.sync_copy(xt_hbm.at[wid], idx_bm)

        def start(c, p):
            cbase = pl.multiple_of(base + c * CH, CH)
            for b in range(BATCH):
                pltpu.make_async_copy(
                    table_hbm.at[
                        idx_bm.at[b, pl.ds(pl.multiple_of(c * CH_S, CH_S), CH_S)]
                    ],
                    rows[p].at[pl.ds(b * CH_S, CH_S)],
                    sg[p],
                ).start()
            pltpu.make_async_copy(
                pe_hbm.at[pl.ds(pl.multiple_of((cbase // BATCH) * D, CH_S * D),
                                CH_S * D)],
                pev[p], sp[p],
            ).start()

        def wait_in(p):
            for b in range(BATCH):
                pltpu.make_async_copy(
                    table_hbm.at[idx_bm.at[b, pl.ds(0, CH_S)]],
                    rows[p].at[pl.ds(b * CH_S, CH_S)],
                    sg[p],
                ).wait()
            pltpu.make_async_copy(
                pe_hbm.at[pl.ds(0, CH_S * D)], pev[p], sp[p]
            ).wait()

        def wait_out(p):
            pltpu.make_async_copy(
                outv[p], out_hbm.at[pl.ds(0, CH_S)], so[p]
            ).wait()

        def writeback(c, p):
            sbase = pl.multiple_of((base + c * CH) // BATCH, CH_S)
            pltpu.make_async_copy(
                outv[p], out_hbm.at[pl.ds(sbase, CH_S)], so[p]
            ).start()

        def compute(rows_v, pe_v, out_v):
            def s_body(si, carry2):
                @plsc.parallel_loop(0, NT * NJ, unroll=8)
                def d_body(di):
                    doff = di * LANES
                    ti = di >> 3
                    jcol = (di & 7) * LANES
                    pvec = pe_v[pl.ds(si * D + doff, LANES)]
                    for b in range(BATCH):
                        out_v[si, ti, b, pl.ds(jcol, LANES)] = (
                            rows_v[b * CH_S + si, pl.ds(doff, LANES)] * SCALE
                            + pvec
                        )

                return carry2

            lax.fori_loop(0, CH_S, s_body, 0)

        start(0, 0)
        for c in (0, 1):
            p = c & 1
            start(c + 1, 1 - p)
            wait_in(p)
            compute(rows[p], pev[p], outv[p])
            writeback(c, p)

        def pair_body(i, carry):
            c0 = i * 2
            for p in (0, 1):
                c = c0 + p

                @pl.when(c + 1 < n_ch)
                def _():
                    start(c + 1, 1 - p)

                wait_in(p)
                wait_out(p)
                compute(rows[p], pev[p], outv[p])
                writeback(c, p)
            return carry

        lax.fori_loop(1, n_ch // 2, pair_body, 0)
        wait_out(0)
        wait_out(1)

    return k


def kernel(x, table, pe):
    seq, batch = x.shape
    D = table.shape[1]
    x3 = x.reshape(seq // TILE_W, TILE_W, batch).transpose(0, 2, 1)
    pef = pe.reshape(pe.shape[0] * pe.shape[2])
    out4 = _make_kernel(seq, batch, table.shape[0], D)(x3, table, pef)
    return out4.transpose(0, 2, 1, 3).reshape(seq, batch, D)


# speedup vs baseline: 2.8346x; 1.0077x over previous
import functools
import math

import jax
import jax.numpy as jnp
from jax import lax
from jax.experimental import pallas as pl
from jax.experimental.pallas import tpu as pltpu
from jax.experimental.pallas import tpu_sc as plsc

D_MODEL = 768
SCALE = math.sqrt(float(D_MODEL))
LANES = 16
TILE_W = 128


@functools.cache
def _make_kernel(SEQ, BATCH, V, D):
    B = SEQ * BATCH
    info = plsc.get_sparse_core_info()
    NC, NS = info.num_cores, info.num_subcores
    NW = NC * NS
    b_per_w = B // NW
    s_per_w = b_per_w // BATCH
    CH = 32
    n_ch = b_per_w // CH
    CH_S = CH // BATCH
    NT = D // TILE_W
    NJ = TILE_W // LANES

    mesh = plsc.VectorSubcoreMesh(core_axis_name="c", subcore_axis_name="s")

    @functools.partial(
        pl.kernel,
        mesh=mesh,
        out_type=jax.ShapeDtypeStruct((SEQ, NT, BATCH, TILE_W), jnp.float32),
        scratch_types=[
            pltpu.VMEM((BATCH, s_per_w), jnp.int32),
            pltpu.VMEM((CH, D), jnp.float32),
            pltpu.VMEM((CH, D), jnp.float32),
            pltpu.VMEM((CH_S, NT, BATCH, TILE_W), jnp.float32),
            pltpu.VMEM((CH_S, NT, BATCH, TILE_W), jnp.float32),
            pltpu.VMEM((CH_S * D,), jnp.float32),
            pltpu.VMEM((CH_S * D,), jnp.float32),
            pltpu.SemaphoreType.DMA,
            pltpu.SemaphoreType.DMA,
            pltpu.SemaphoreType.DMA,
            pltpu.SemaphoreType.DMA,
            pltpu.SemaphoreType.DMA,
            pltpu.SemaphoreType.DMA,
        ],
    )
    def k(xt_hbm, table_hbm, pe_hbm, out_hbm,
          idx_bm, rows0, rows1, ov0, ov1, pe0, pe1,
          sg0, sg1, sp0, sp1, so0, so1):
        rows = [rows0, rows1]
        outv = [ov0, ov1]
        pev = [pe0, pe1]
        sg = [sg0, sg1]
        sp = [sp0, sp1]
        so = [so0, so1]
        wid = lax.axis_index("s") * NC + lax.axis_index("c")
        base = wid * b_per_w

        pltpu.sync_copy(xt_hbm.at[wid], idx_bm)

        def start(c, p):
            cbase = pl.multiple_of(base + c * CH, CH)
            pltpu.make_async_copy(
                pe_hbm.at[pl.ds(pl.multiple_of((cbase // BATCH) * D, CH_S * D),
                                CH_S * D)],
                pev[p], sp[p],
            ).start()
            for b in range(BATCH):
                pltpu.make_async_copy(
                    table_hbm.at[
                        idx_bm.at[b, pl.ds(pl.multiple_of(c * CH_S, CH_S), CH_S)]
                    ],
                    rows[p].at[pl.ds(b * CH_S, CH_S)],
                    sg[p],
                ).start()

        def wait_in(p):
            for b in range(BATCH):
                pltpu.make_async_copy(
                    table_hbm.at[idx_bm.at[b, pl.ds(0, CH_S)]],
                    rows[p].at[pl.ds(b * CH_S, CH_S)],
                    sg[p],
                ).wait()
            pltpu.make_async_copy(
                pe_hbm.at[pl.ds(0, CH_S * D)], pev[p], sp[p]
            ).wait()

        def wait_out(p):
            pltpu.make_async_copy(
                outv[p], out_hbm.at[pl.ds(0, CH_S)], so[p]
            ).wait()

        def writeback(c, p):
            sbase = pl.multiple_of((base + c * CH) // BATCH, CH_S)
            pltpu.make_async_copy(
                outv[p], out_hbm.at[pl.ds(sbase, CH_S)], so[p]
            ).start()

        def compute(rows_v, pe_v, out_v):
            def s_body(si, carry2):
                @plsc.parallel_loop(0, NT * NJ, unroll=8)
                def d_body(di):
                    doff = di * LANES
                    ti = di >> 3
                    jcol = (di & 7) * LANES
                    pvec = pe_v[pl.ds(si * D + doff, LANES)]
                    for b in range(BATCH):
                        out_v[si, ti, b, pl.ds(jcol, LANES)] = (
                            rows_v[b * CH_S + si, pl.ds(doff, LANES)] * SCALE
                            + pvec
                        )

                return carry2

            lax.fori_loop(0, CH_S, s_body, 0)

        start(0, 0)
        for c in (0, 1):
            p = c & 1
            start(c + 1, 1 - p)
            wait_in(p)
            compute(rows[p], pev[p], outv[p])
            writeback(c, p)

        def pair_body(i, carry):
            c0 = i * 2
            for p in (0, 1):
                c = c0 + p

                @pl.when(c + 1 < n_ch)
                def _():
                    start(c + 1, 1 - p)

                wait_in(p)
                wait_out(p)
                compute(rows[p], pev[p], outv[p])
                writeback(c, p)
            return carry

        lax.fori_loop(1, n_ch // 2, pair_body, 0)
        wait_out(0)
        wait_out(1)

    return k


def kernel(x, table, pe):
    seq, batch = x.shape
    D = table.shape[1]
    x3 = x.reshape(seq // TILE_W, TILE_W, batch).transpose(0, 2, 1)
    pef = pe.reshape(pe.shape[0] * pe.shape[2])
    out4 = _make_kernel(seq, batch, table.shape[0], D)(x3, table, pef)
    return out4.transpose(0, 2, 1, 3).reshape(seq, batch, D)
